# baseline (speedup 1.0000x reference)
import functools
from math import sqrt

import jax
import jax.numpy as jnp
from jax import lax
from jax.experimental import pallas as pl
from jax.experimental.pallas import tpu as pltpu
from jax.experimental.pallas import tpu_sc as plsc

D_MODEL = 768
SCALE = sqrt(D_MODEL)
LANES = 16
VECS_PER_ROW = D_MODEL // LANES

NUM_CORES = 2
NUM_SUBCORES = 16
NW = NUM_CORES * NUM_SUBCORES

B = 4 * 4096
B_PER_W = B // NW
CH = 64
NCH = B_PER_W // CH
X_COLS = 4096
W_PER_ROW = X_COLS // B_PER_W

_mesh = plsc.VectorSubcoreMesh(
    core_axis_name="c", subcore_axis_name="s", num_cores=NUM_CORES
)


@functools.partial(
    pl.kernel,
    out_type=jax.ShapeDtypeStruct((B, D_MODEL), jnp.float32),
    mesh=_mesh,
    scratch_types=[
        pltpu.VMEM((B_PER_W,), jnp.int32),
        pltpu.VMEM((CH, D_MODEL), jnp.float32),
        pltpu.VMEM((CH, D_MODEL), jnp.float32),
        pltpu.SemaphoreType.DMA,
        pltpu.SemaphoreType.DMA,
    ],
)
def _emb_lookup(idx_hbm, table_hbm, out_hbm, idx_v, buf0, buf1, gsemA,
                gsemB):
    wid = lax.axis_index("s") * NUM_CORES + lax.axis_index("c")
    base = wid * B_PER_W

    w_row = wid // W_PER_ROW
    w_col = (wid % W_PER_ROW) * B_PER_W
    pltpu.sync_copy(idx_hbm.at[w_row, pl.ds(w_col, B_PER_W)], idx_v)

    bufs = (buf0, buf1)

    HALF = CH // 2
    gsems = (gsemA, gsemB)

    def gather(j, buf):
        for p in range(2):
            pltpu.async_copy(
                table_hbm.at[idx_v.at[pl.ds(j * CH + p * HALF, HALF)]],
                buf.at[pl.ds(p * HALF, HALF)], gsems[p],
            )

    gather(0, buf0)
    gather(1, buf1)

    @pl.loop(0, NCH, step=2)
    def _chunk_pair(j):
        for b, buf in enumerate(bufs):
            jj = j + b
            for p in range(2):
                pltpu.make_async_copy(
                    table_hbm.at[idx_v.at[pl.ds(0, HALF)]],
                    buf.at[pl.ds(p * HALF, HALF)], gsems[p],
                ).wait()

                @pl.loop(p * HALF, (p + 1) * HALF)
                def _scale_row(r, buf=buf):
                    for c in range(VECS_PER_ROW):
                        sl = (r, pl.ds(c * LANES, LANES))
                        buf[sl] = buf[sl] * SCALE

            pltpu.sync_copy(buf, out_hbm.at[pl.ds(base + jj * CH, CH)])

            @pl.when(jj + 2 < NCH)
            def _prefetch(jj=jj, buf=buf):
                gather(jj + 2, buf)


def kernel(x, table):
    out = _emb_lookup(x, table)
    return out.reshape(x.shape[0], x.shape[1], D_MODEL)


# speedup vs baseline: 1.0406x; 1.0406x over previous
import functools
from math import sqrt

import jax
import jax.numpy as jnp
from jax import lax
from jax.experimental import pallas as pl
from jax.experimental.pallas import tpu as pltpu
from jax.experimental.pallas import tpu_sc as plsc

D_MODEL = 768
SCALE = sqrt(D_MODEL)
LANES = 16
VECS_PER_ROW = D_MODEL // LANES

NUM_CORES = 2
NUM_SUBCORES = 16
NW = NUM_CORES * NUM_SUBCORES

B = 4 * 4096
B_PER_W = B // NW
CH = 64
NCH = B_PER_W // CH
X_COLS = 4096
W_PER_ROW = X_COLS // B_PER_W

_mesh = plsc.VectorSubcoreMesh(
    core_axis_name="c", subcore_axis_name="s", num_cores=NUM_CORES
)


@functools.partial(
    pl.kernel,
    out_type=jax.ShapeDtypeStruct((B, D_MODEL), jnp.float32),
    mesh=_mesh,
    scratch_types=[
        pltpu.VMEM((B_PER_W,), jnp.int32),
        pltpu.VMEM((CH, D_MODEL), jnp.float32),
        pltpu.VMEM((CH, D_MODEL), jnp.float32),
        pltpu.SemaphoreType.DMA,
    ],
)
def _emb_lookup(idx_hbm, table_hbm, out_hbm, idx_v, buf0, buf1, gsem):
    wid = lax.axis_index("s") * NUM_CORES + lax.axis_index("c")
    base = wid * B_PER_W

    w_row = wid // W_PER_ROW
    w_col = (wid % W_PER_ROW) * B_PER_W
    pltpu.sync_copy(idx_hbm.at[w_row, pl.ds(w_col, B_PER_W)], idx_v)

    bufs = (buf0, buf1)

    def gather(j, buf):
        half = CH // 2
        for p in range(2):
            pltpu.async_copy(
                table_hbm.at[idx_v.at[pl.ds(j * CH + p * half, half)]],
                buf.at[pl.ds(p * half, half)], gsem,
            )

    gather(0, buf0)
    gather(1, buf1)

    @pl.loop(0, NCH, step=2)
    def _chunk_pair(j):
        for b, buf in enumerate(bufs):
            jj = j + b
            pltpu.make_async_copy(
                table_hbm.at[idx_v.at[pl.ds(0, CH)]], buf, gsem
            ).wait()

            @pl.loop(0, CH)
            def _scale_row(r, buf=buf):
                for c in range(VECS_PER_ROW):
                    sl = (r, pl.ds(c * LANES, LANES))
                    buf[sl] = buf[sl] * SCALE

            pltpu.sync_copy(buf, out_hbm.at[pl.ds(base + jj * CH, CH)])

            @pl.when(jj + 2 < NCH)
            def _prefetch(jj=jj, buf=buf):
                gather(jj + 2, buf)


def kernel(x, table):
    out = _emb_lookup(x.astype(jnp.int32), table)
    return out.reshape(x.shape[0], x.shape[1], D_MODEL)
